# baseline (speedup 1.0000x reference)
import functools

import jax
import jax.numpy as jnp
from jax import lax
from jax.experimental import pallas as pl
from jax.experimental.pallas import tpu as pltpu
from jax.experimental.pallas import tpu_sc as plsc

NUM_BINS = 100000
DIM = 32
B = 16384
N = 100

NC = 2
NS = 16
NW = NC * NS

KB = 16
NBUF = 2
B_PER_W = B // NW
S = B_PER_W // KB

_mesh = plsc.VectorSubcoreMesh(core_axis_name="c", subcore_axis_name="s")


@functools.partial(
    pl.kernel,
    out_type=jax.ShapeDtypeStruct((B, 128, DIM), jnp.float32),
    mesh=_mesh,
    scratch_types=[
        pltpu.VMEM((NBUF, KB, N), jnp.int32),
        pltpu.VMEM((NBUF, KB, N, DIM), jnp.float32),
        [pltpu.SemaphoreType.DMA] * NBUF,
        [pltpu.SemaphoreType.DMA] * NBUF,
    ],
    compiler_params=pltpu.CompilerParams(use_tc_tiling_on_sc=False),
)
def _emb_lookup(table_hbm, idx_hbm, out_hbm, idx_v, rows_v, gsem, osem):
    wid = lax.axis_index("s") * NC + lax.axis_index("c")
    batch_base = wid * B_PER_W

    def fire_gathers(b):
        return [
            pltpu.async_copy(
                table_hbm.at[idx_v.at[b].at[i]],
                rows_v.at[b].at[i],
                gsem[b],
            )
            for i in range(KB)
        ]

    for b in range(NBUF):
        pltpu.sync_copy(idx_hbm.at[pl.ds(batch_base + b * KB, KB)], idx_v.at[b])
        fire_gathers(b)

    def body(go, carry):
        for b in range(NBUF):
            g = go * NBUF + b
            b0 = batch_base + g * KB
            for i in range(KB):
                pltpu.make_async_copy(
                    table_hbm.at[idx_v.at[b].at[i]],
                    rows_v.at[b].at[i],
                    gsem[b],
                ).wait()
            ocp = pltpu.async_copy(
                rows_v.at[b], out_hbm.at[pl.ds(b0, KB), pl.ds(0, N)], osem[b]
            )
            gn = g + NBUF

            @pl.when(gn < S)
            def _():
                pltpu.sync_copy(
                    idx_hbm.at[pl.ds(batch_base + gn * KB, KB)], idx_v.at[b]
                )

            ocp.wait()

            @pl.when(gn < S)
            def _():
                fire_gathers(b)

        return carry

    lax.fori_loop(0, S // NBUF, body, 0)


def kernel(x, weight):
    return _emb_lookup(weight, x.astype(jnp.int32))[:, :N, :]


# speedup vs baseline: 1.0998x; 1.0998x over previous
import functools

import jax
import jax.numpy as jnp
from jax import lax
from jax.experimental import pallas as pl
from jax.experimental.pallas import tpu as pltpu
from jax.experimental.pallas import tpu_sc as plsc

NUM_BINS = 100000
DIM = 32
B = 16384
N = 100

NC = 2
NS = 16
NW = NC * NS

B_PER_W = B // NW
NBB = B_PER_W // 128
NDB = DIM // 8
NBUF = 2

_mesh = plsc.VectorSubcoreMesh(core_axis_name="c", subcore_axis_name="s")


@functools.partial(
    pl.kernel,
    out_type=jax.ShapeDtypeStruct((N, NDB, B // 128, 8, 128), jnp.float32),
    mesh=_mesh,
    scratch_types=[
        pltpu.VMEM((NBUF, B_PER_W), jnp.int32),
        pltpu.VMEM((NBUF, B_PER_W, DIM), jnp.float32),
        pltpu.VMEM((NBUF, NDB, NBB, 8, 128), jnp.float32),
        [pltpu.SemaphoreType.DMA] * NBUF,
        [pltpu.SemaphoreType.DMA] * NBUF,
    ],
    compiler_params=pltpu.CompilerParams(
        use_tc_tiling_on_sc=False, needs_layout_passes=False
    ),
)
def _emb_lookup(table_hbm, xt_hbm, out_hbm, idx_v, rows_v, trans_v, gsem, osem):
    wid = lax.axis_index("s") * NC + lax.axis_index("c")
    b0 = wid * B_PER_W
    bb0 = wid * NBB
    iota16 = lax.iota(jnp.int32, 16)

    def load_idx(s, n):
        pltpu.sync_copy(xt_hbm.at[n, pl.ds(b0, B_PER_W)], idx_v.at[s])

    def fire_gathers(s):
        for j in range(NBB):
            pltpu.async_copy(
                table_hbm.at[idx_v.at[s].at[pl.ds(j * 128, 128)]],
                rows_v.at[s].at[pl.ds(j * 128, 128)],
                gsem[s],
            )

    def wait_gathers(s):
        for j in range(NBB):
            pltpu.make_async_copy(
                table_hbm.at[idx_v.at[s].at[pl.ds(j * 128, 128)]],
                rows_v.at[s].at[pl.ds(j * 128, 128)],
                gsem[s],
            ).wait()

    def transpose(s):
        for dblk in range(NDB):
            for bb in range(NBB):
                for din in range(8):
                    cidx = jnp.full((16,), dblk * 8 + din, jnp.int32)
                    for c in range(8):
                        ridx = iota16 + (bb * 128 + c * 16)
                        vals = plsc.load_gather(rows_v.at[s], [ridx, cidx])
                        trans_v[s, dblk, bb, din, pl.ds(c * 16, 16)] = vals

    def fire_out(s, n):
        for dblk in range(NDB):
            pltpu.async_copy(
                trans_v.at[s].at[dblk],
                out_hbm.at[n, dblk, pl.ds(bb0, NBB)],
                osem[s],
            )

    def wait_out(s, n):
        for dblk in range(NDB):
            pltpu.make_async_copy(
                trans_v.at[s].at[dblk],
                out_hbm.at[n, dblk, pl.ds(bb0, NBB)],
                osem[s],
            ).wait()

    load_idx(0, 0)
    fire_gathers(0)
    load_idx(1, 1)

    def body(g, carry):
        for s in range(NBUF):
            n = g * NBUF + s
            wait_gathers(s)

            @pl.when(n + 1 < N)
            def _():
                fire_gathers(1 - s)

            @pl.when(n >= NBUF)
            def _():
                wait_out(s, n - NBUF)

            transpose(s)
            fire_out(s, n)

            @pl.when(n + NBUF < N)
            def _():
                load_idx(s, n + NBUF)

        return carry

    lax.fori_loop(0, N // NBUF, body, 0)
    wait_out(0, N - 2)
    wait_out(1, N - 1)


def kernel(x, weight):
    o5 = _emb_lookup(weight, x.T.astype(jnp.int32))
    return o5.transpose(2, 4, 0, 1, 3).reshape(B, N, DIM)


# speedup vs baseline: 1.6291x; 1.4813x over previous
import functools

import jax
import jax.numpy as jnp
from jax import lax
from jax.experimental import pallas as pl
from jax.experimental.pallas import tpu as pltpu
from jax.experimental.pallas import tpu_sc as plsc

NUM_BINS = 100000
DIM = 32
B = 16384
N = 100

NC = 2
NS = 16
NW = NC * NS

B_PER_W = B // NW
NBB = B_PER_W // 128
NDB = DIM // 8
NBUF = 2

_mesh = plsc.VectorSubcoreMesh(core_axis_name="c", subcore_axis_name="s")


@functools.partial(
    pl.kernel,
    out_type=jax.ShapeDtypeStruct((N, NDB, B // 128, 8, 128), jnp.float32),
    mesh=_mesh,
    scratch_types=[
        pltpu.VMEM((NBUF, B_PER_W), jnp.int32),
        pltpu.VMEM((NBUF, B_PER_W, DIM), jnp.float32),
        pltpu.VMEM((NBUF, NDB, NBB, 8, 128), jnp.float32),
        [pltpu.SemaphoreType.DMA] * NBUF,
        [pltpu.SemaphoreType.DMA] * NBUF,
    ],
    compiler_params=pltpu.CompilerParams(
        use_tc_tiling_on_sc=False, needs_layout_passes=False
    ),
)
def _emb_lookup(table_hbm, xt_hbm, out_hbm, idx_v, rows_v, trans_v, gsem, osem):
    wid = lax.axis_index("s") * NC + lax.axis_index("c")
    b0 = wid * B_PER_W
    bb0 = wid * NBB
    iota16 = lax.iota(jnp.int32, 16)

    def load_idx(s, n):
        pltpu.sync_copy(xt_hbm.at[n, pl.ds(b0, B_PER_W)], idx_v.at[s])

    def fire_gathers(s):
        for j in range(NBB):
            pltpu.async_copy(
                table_hbm.at[idx_v.at[s].at[pl.ds(j * 128, 128)]],
                rows_v.at[s].at[pl.ds(j * 128, 128)],
                gsem[s],
            )

    def wait_gathers(s):
        for j in range(NBB):
            pltpu.make_async_copy(
                table_hbm.at[idx_v.at[s].at[pl.ds(j * 128, 128)]],
                rows_v.at[s].at[pl.ds(j * 128, 128)],
                gsem[s],
            ).wait()

    def transpose(s):
        @plsc.parallel_loop(0, NDB * NBB * 8 * 8, unroll=8)
        def _(t):
            dblk = t >> 8
            bb = (t >> 6) & (NBB - 1)
            din = (t >> 3) & 7
            c = t & 7
            ridx = iota16 + (bb * 128 + c * 16)
            cidx = jnp.full((16,), dblk * 8 + din, jnp.int32)
            vals = plsc.load_gather(rows_v.at[s], [ridx, cidx])
            trans_v[s, dblk, bb, din, pl.ds(c * 16, 16)] = vals

    def fire_out(s, n):
        for dblk in range(NDB):
            pltpu.async_copy(
                trans_v.at[s].at[dblk],
                out_hbm.at[n, dblk, pl.ds(bb0, NBB)],
                osem[s],
            )

    def wait_out(s, n):
        for dblk in range(NDB):
            pltpu.make_async_copy(
                trans_v.at[s].at[dblk],
                out_hbm.at[n, dblk, pl.ds(bb0, NBB)],
                osem[s],
            ).wait()

    load_idx(0, 0)
    fire_gathers(0)
    load_idx(1, 1)

    def body(g, carry):
        for s in range(NBUF):
            n = g * NBUF + s
            wait_gathers(s)

            @pl.when(n + 1 < N)
            def _():
                fire_gathers(1 - s)

            @pl.when(n >= NBUF)
            def _():
                wait_out(s, n - NBUF)

            transpose(s)
            fire_out(s, n)

            @pl.when(n + NBUF < N)
            def _():
                load_idx(s, n + NBUF)

        return carry

    lax.fori_loop(0, N // NBUF, body, 0)
    wait_out(0, N - 2)
    wait_out(1, N - 1)


def kernel(x, weight):
    o5 = _emb_lookup(weight, x.T.astype(jnp.int32))
    return o5.transpose(2, 4, 0, 1, 3).reshape(B, N, DIM)


# speedup vs baseline: 2.2163x; 1.3604x over previous
import functools

import jax
import jax.numpy as jnp
from jax import lax
from jax.experimental import pallas as pl
from jax.experimental.pallas import tpu as pltpu
from jax.experimental.pallas import tpu_sc as plsc

NUM_BINS = 100000
DIM = 32
B = 16384
N = 100

NC = 2
NS = 16
NW = NC * NS

B_PER_W = B // NW
NBB = B_PER_W // 128
NDB = DIM // 8
NBUF = 2

_mesh = plsc.VectorSubcoreMesh(core_axis_name="c", subcore_axis_name="s")


@functools.partial(
    pl.kernel,
    out_type=jax.ShapeDtypeStruct((N, NDB, B // 128, 8, 128), jnp.float32),
    mesh=_mesh,
    scratch_types=[
        pltpu.VMEM((NBUF, B_PER_W), jnp.int32),
        pltpu.VMEM((NBUF, B_PER_W, DIM), jnp.float32),
        pltpu.VMEM((NBUF, NDB, NBB, 8, 128), jnp.float32),
        [pltpu.SemaphoreType.DMA] * NBUF,
        [pltpu.SemaphoreType.DMA] * NBUF,
    ],
    compiler_params=pltpu.CompilerParams(
        use_tc_tiling_on_sc=False, needs_layout_passes=False
    ),
)
def _emb_lookup(table_hbm, xt_hbm, out_hbm, idx_v, rows_v, trans_v, gsem, osem):
    wid = lax.axis_index("s") * NC + lax.axis_index("c")
    b0 = wid * B_PER_W
    bb0 = wid * NBB
    iota16 = lax.iota(jnp.int32, 16)

    def load_idx(s, n):
        pltpu.sync_copy(xt_hbm.at[n, pl.ds(b0, B_PER_W)], idx_v.at[s])

    def fire_gathers(s):
        for j in range(NBB):
            pltpu.async_copy(
                table_hbm.at[idx_v.at[s].at[pl.ds(j * 128, 128)]],
                rows_v.at[s].at[pl.ds(j * 128, 128)],
                gsem[s],
            )

    def wait_gathers(s):
        for j in range(NBB):
            pltpu.make_async_copy(
                table_hbm.at[idx_v.at[s].at[pl.ds(j * 128, 128)]],
                rows_v.at[s].at[pl.ds(j * 128, 128)],
                gsem[s],
            ).wait()

    def transpose(s):
        @plsc.parallel_loop(0, NDB * NBB * 8, unroll=4)
        def _(u):
            dblk = u >> 5
            bb = (u >> 3) & (NBB - 1)
            din = u & 7
            cidx = jnp.full((16,), dblk * 8 + din, jnp.int32)
            rbase = bb * 128
            for c in range(8):
                ridx = (iota16 + c * 16) + rbase
                vals = plsc.load_gather(rows_v.at[s], [ridx, cidx])
                trans_v[s, dblk, bb, din, pl.ds(c * 16, 16)] = vals

    def fire_out(s, n):
        for dblk in range(NDB):
            pltpu.async_copy(
                trans_v.at[s].at[dblk],
                out_hbm.at[n, dblk, pl.ds(bb0, NBB)],
                osem[s],
            )

    def wait_out(s, n):
        for dblk in range(NDB):
            pltpu.make_async_copy(
                trans_v.at[s].at[dblk],
                out_hbm.at[n, dblk, pl.ds(bb0, NBB)],
                osem[s],
            ).wait()

    load_idx(0, 0)
    fire_gathers(0)
    load_idx(1, 1)

    def body(g, carry):
        for s in range(NBUF):
            n = g * NBUF + s
            wait_gathers(s)

            @pl.when(n + 1 < N)
            def _():
                fire_gathers(1 - s)

            @pl.when(n >= NBUF)
            def _():
                wait_out(s, n - NBUF)

            transpose(s)
            fire_out(s, n)

            @pl.when(n + NBUF < N)
            def _():
                load_idx(s, n + NBUF)

        return carry

    lax.fori_loop(0, N // NBUF, body, 0)
    wait_out(0, N - 2)
    wait_out(1, N - 1)


def kernel(x, weight):
    o5 = _emb_lookup(weight, x.T.astype(jnp.int32))
    return o5.transpose(2, 4, 0, 1, 3).reshape(B, N, DIM)
